# Initial kernel scaffold; baseline (speedup 1.0000x reference)
import functools

import jax
import jax.numpy as jnp
from jax import lax
from jax.experimental import pallas as pl
from jax.experimental.pallas import tpu as pltpu
from jax.experimental.pallas import tpu_sc as plsc

B = 32
KX = 30
NTRI = 512
NW = 32
TPW = NTRI // NW
RPW = 3 * TPW
L = 16

_mesh = plsc.VectorSubcoreMesh(core_axis_name="c", subcore_axis_name="s")


def _nsqrt(x):
    i = lax.bitcast_convert_type(x, jnp.int32)
    y = lax.bitcast_convert_type(
        lax.shift_right_arithmetic(i, 1) + jnp.int32(0x1FBD1DF5), jnp.float32)
    for _ in range(3):
        y = 0.5 * (y + x / y)
    return y


@functools.partial(
    pl.kernel,
    out_type=jax.ShapeDtypeStruct((2, B, L), jnp.float32),
    mesh=_mesh,
    scratch_types=[
        pltpu.VMEM((RPW,), jnp.int32),
        pltpu.VMEM((RPW,), jnp.int32),
        pltpu.VMEM((RPW, 32), jnp.float32),
        pltpu.VMEM((RPW,), jnp.float32),
        pltpu.VMEM((B, KX), jnp.float32),
        pltpu.VMEM((B, RPW), jnp.float32),
        pltpu.VMEM((B, L), jnp.float32),
        pltpu.VMEM((16, 2, L), jnp.float32),
        pltpu.VMEM((2, L), jnp.float32),
        pltpu.VMEM_SHARED((16, B, L), jnp.float32),
        pltpu.SemaphoreType.DMA,
        pltpu.SemaphoreType.DMA,
    ],
)
def _sc_loss(ev2_hbm, rlt_hbm, x_hbm, tgt_hbm, mean_hbm, out_hbm,
             ridx_v, eidx_v, g_v, mean_v, x_v, t_v, part_v, red_v, obuf_v,
             shared, sem_g, sem_m):
    c = lax.axis_index("c")
    s = lax.axis_index("s")
    g = c * 16 + s
    base = g * RPW

    pltpu.sync_copy(rlt_hbm.at[pl.ds(base, RPW)], ridx_v)
    for ch in range(RPW // L):
        eidx_v[pl.ds(ch * L, L)] = ridx_v[pl.ds(ch * L, L)] * 2
    cp_g = pltpu.async_copy(ev2_hbm.at[eidx_v], g_v, sem_g)
    cp_m = pltpu.async_copy(mean_hbm.at[ridx_v], mean_v, sem_m)
    pltpu.sync_copy(x_hbm, x_v)
    pltpu.sync_copy(tgt_hbm.at[:, pl.ds(base, RPW)], t_v)
    cp_g.wait()
    cp_m.wait()

    iota = lax.iota(jnp.int32, L)
    row3 = [iota * 3 + j for j in range(3)]
    mt = [plsc.load_gather(mean_v, [row3[j]]) for j in range(3)]
    tcol = iota * 3

    for bc in range(B // 8):
        acc = [[mt[j] for _ in range(8)] for j in range(3)]
        for k in range(30):
            kf = jnp.full((L,), k, jnp.int32)
            col = [plsc.load_gather(g_v, [row3[j], kf]) for j in range(3)]
            for b8 in range(8):
                b = bc * 8 + b8
                xv = plsc.load_gather(
                    x_v, [jnp.full((L,), b, jnp.int32), kf])
                for j in range(3):
                    acc[j][b8] = acc[j][b8] + col[j] * xv
        for b8 in range(8):
            b = bc * 8 + b8
            tt = plsc.load_gather(
                t_v, [jnp.full((L,), b, jnp.int32), tcol])
            d0 = acc[0][b8] - tt
            d1 = acc[1][b8] - tt
            d2 = acc[2][b8] - tt
            dist = _nsqrt(d0 * d0 + d1 * d1 + d2 * d2)
            part_v[b, :] = jnp.broadcast_to(jnp.sum(dist), (L,))

    pltpu.sync_copy(part_v, shared.at[s])
    plsc.subcore_barrier()
    b0 = s * 2
    for i in range(16):
        pltpu.sync_copy(shared.at[i, pl.ds(b0, 2)], red_v.at[i])
    for p in range(2):
        acc_r = red_v[0, p, :]
        for i in range(1, 16):
            acc_r = acc_r + red_v[i, p, :]
        obuf_v[p, :] = acc_r
    pltpu.sync_copy(obuf_v, out_hbm.at[c, pl.ds(b0, 2)])


def kernel(output, target, eigenVectors, mean, indices, outline, rightLineIdxs):
    ev2 = eigenVectors.reshape(-1, 32)
    part = _sc_loss(ev2, rightLineIdxs, output, target, mean)
    return (part[0] + part[1])[:, 0]


# speedup vs baseline: 2.2595x; 2.2595x over previous
import functools

import jax
import jax.numpy as jnp
from jax import lax
from jax.experimental import pallas as pl
from jax.experimental.pallas import tpu as pltpu
from jax.experimental.pallas import tpu_sc as plsc

B = 32
KX = 30
NTRI = 512
NW = 32
TPW = NTRI // NW
RPW = 3 * TPW
L = 16

_mesh = plsc.VectorSubcoreMesh(core_axis_name="c", subcore_axis_name="s")


def _nsqrt(x):
    i = lax.bitcast_convert_type(x, jnp.int32)
    y = lax.bitcast_convert_type(
        lax.shift_right_arithmetic(i, 1) + jnp.int32(0x1FBD1DF5), jnp.float32)
    for _ in range(3):
        y = 0.5 * (y + x / y)
    return y


@functools.partial(
    pl.kernel,
    out_type=jax.ShapeDtypeStruct((2, B, L), jnp.float32),
    mesh=_mesh,
    compiler_params=pltpu.CompilerParams(needs_layout_passes=False),
    scratch_types=[
        pltpu.VMEM((RPW,), jnp.int32),
        pltpu.VMEM((RPW,), jnp.int32),
        pltpu.VMEM((RPW,), jnp.int32),
        pltpu.VMEM((RPW, 128), jnp.float32),
        pltpu.VMEM((RPW, 128), jnp.float32),
        pltpu.VMEM((8 + B * KX, ), jnp.float32),
        pltpu.VMEM((2 * B,), jnp.int32),
        pltpu.VMEM((2 * B, 128), jnp.float32),
        pltpu.VMEM((B, L), jnp.float32),
        pltpu.VMEM((16, B, L), jnp.float32),
        pltpu.VMEM((B, L), jnp.float32),
        pltpu.HBM((2, 16, B, L), jnp.float32),
        pltpu.SemaphoreType.DMA,
        pltpu.SemaphoreType.DMA,
        pltpu.SemaphoreType.DMA,
    ],
)
def _sc_loss(ev2_hbm, rlt_hbm, x_hbm, tgt_hbm, mean_hbm, out_hbm,
             ridx_v, eidx_v, midx_v, g_v, mean_v, x_v, tidx_v, t_v, part_v,
             red_v, osum_v, stage, sem_g, sem_m, sem_t):
    c = lax.axis_index("c")
    s = lax.axis_index("s")
    g = c * 16 + s
    base = g * RPW

    iota = lax.iota(jnp.int32, L)

    pltpu.sync_copy(rlt_hbm.at[pl.ds(base, RPW)], ridx_v)
    for ch in range(RPW // L):
        r = ridx_v[pl.ds(ch * L, L)]
        eidx_v[pl.ds(ch * L, L)] = lax.shift_right_logical(r, 1)
        midx_v[pl.ds(ch * L, L)] = lax.shift_right_logical(r, 7)
    trow = lax.shift_right_logical(base, 7)
    for ch in range(2 * B // L):
        n = iota + ch * L
        row = 12 * lax.shift_right_logical(n, 1) + trow + (n & 1)
        tidx_v[pl.ds(ch * L, L)] = jnp.minimum(row, 383)
    cp_g = pltpu.async_copy(ev2_hbm.at[eidx_v], g_v, sem_g)
    cp_m = pltpu.async_copy(mean_hbm.at[midx_v], mean_v, sem_m)
    cp_t = pltpu.async_copy(tgt_hbm.at[tidx_v], t_v, sem_t)
    pltpu.sync_copy(x_hbm, x_v.at[pl.ds(8, B * KX)])
    cp_g.wait()
    cp_m.wait()
    cp_t.wait()

    row3 = [iota * 3 + j for j in range(3)]
    rj = [plsc.load_gather(ridx_v, [row3[j]]) for j in range(3)]
    mt = [plsc.load_gather(mean_v, [row3[j], rj[j] & 127]) for j in range(3)]
    par = [(rj[j] & 1) * 64 for j in range(3)]
    toff = (base & 127) + iota * 3
    trow2 = lax.shift_right_logical(toff, 7)
    tcol = toff & 127

    for bc in range(B // 8):
        acc = [[mt[j] for _ in range(8)] for j in range(3)]
        for k in range(30):
            col = [plsc.load_gather(g_v, [row3[j], par[j] + k])
                   for j in range(3)]
            for b8 in range(8):
                b = bc * 8 + b8
                xv = plsc.load_gather(
                    x_v, [jnp.full((L,), 8 + b * KX + k, jnp.int32)])
                for j in range(3):
                    acc[j][b8] = acc[j][b8] + col[j] * xv
        for b8 in range(8):
            b = bc * 8 + b8
            tt = plsc.load_gather(t_v, [trow2 + 2 * b, tcol])
            d0 = acc[0][b8] - tt
            d1 = acc[1][b8] - tt
            d2 = acc[2][b8] - tt
            dist = _nsqrt(d0 * d0 + d1 * d1 + d2 * d2)
            part_v[b, :] = jnp.broadcast_to(jnp.sum(dist), (L,))

    pltpu.sync_copy(part_v, stage.at[c, s])
    plsc.subcore_barrier()

    @pl.when(s == 0)
    def _fold():
        for i in range(16):
            pltpu.sync_copy(stage.at[c, i], red_v.at[i])
        for b in range(B):
            acc_r = red_v[0, b, :]
            for i in range(1, 16):
                acc_r = acc_r + red_v[i, b, :]
            osum_v[b, :] = acc_r
        pltpu.sync_copy(osum_v, out_hbm.at[c])


def kernel(output, target, eigenVectors, mean, indices, outline, rightLineIdxs):
    ev2 = eigenVectors.reshape(-1, 128)
    tgt2 = target.reshape(-1, 128)
    mean2 = mean.reshape(-1, 128)
    part = _sc_loss(ev2, rightLineIdxs, output.reshape(-1), tgt2, mean2)
    return (part[0] + part[1])[:, 0]


# speedup vs baseline: 2.5188x; 1.1148x over previous
import functools

import jax
import jax.numpy as jnp
from jax import lax
from jax.experimental import pallas as pl
from jax.experimental.pallas import tpu as pltpu
from jax.experimental.pallas import tpu_sc as plsc

B = 32
KX = 30
NTRI = 512
NW = 32
TPW = NTRI // NW
RPW = 3 * TPW
L = 16

_mesh = plsc.VectorSubcoreMesh(core_axis_name="c", subcore_axis_name="s")


def _nsqrt(x):
    i = lax.bitcast_convert_type(x, jnp.int32)
    y = lax.bitcast_convert_type(
        lax.shift_right_arithmetic(i, 1) + jnp.int32(0x1FBD1DF5), jnp.float32)
    for _ in range(3):
        y = 0.5 * (y + x / y)
    return y


@functools.partial(
    pl.kernel,
    out_type=jax.ShapeDtypeStruct((2, B, L), jnp.float32),
    mesh=_mesh,
    compiler_params=pltpu.CompilerParams(needs_layout_passes=False),
    scratch_types=[
        pltpu.VMEM((RPW,), jnp.int32),
        pltpu.VMEM((RPW,), jnp.int32),
        pltpu.VMEM((RPW,), jnp.int32),
        pltpu.VMEM((RPW, 128), jnp.float32),
        pltpu.VMEM((RPW, 128), jnp.float32),
        pltpu.VMEM((8 + B * KX, ), jnp.float32),
        pltpu.VMEM((2 * B,), jnp.int32),
        pltpu.VMEM((2 * B, 128), jnp.float32),
        pltpu.VMEM((B, L), jnp.float32),
        pltpu.VMEM((16, B, L), jnp.float32),
        pltpu.VMEM((B, L), jnp.float32),
        pltpu.HBM((2, 16, B, L), jnp.float32),
        pltpu.SemaphoreType.DMA,
        pltpu.SemaphoreType.DMA,
        pltpu.SemaphoreType.DMA,
        pltpu.SemaphoreType.DMA,
    ],
)
def _sc_loss(ev2_hbm, rlt_hbm, x_hbm, tgt_hbm, mean_hbm, out_hbm,
             ridx_v, eidx_v, midx_v, g_v, mean_v, x_v, tidx_v, t_v, part_v,
             red_v, osum_v, stage, sem_g, sem_m, sem_t, sem_x):
    c = lax.axis_index("c")
    s = lax.axis_index("s")
    g = c * 16 + s
    base = g * RPW

    iota = lax.iota(jnp.int32, L)

    cp_r = pltpu.async_copy(rlt_hbm.at[pl.ds(base, RPW)], ridx_v, sem_m)
    cp_x = pltpu.async_copy(x_hbm, x_v.at[pl.ds(8, B * KX)], sem_x)
    trow = lax.shift_right_logical(base, 7)
    for ch in range(2 * B // L):
        n = iota + ch * L
        row = 12 * lax.shift_right_logical(n, 1) + trow + (n & 1)
        tidx_v[pl.ds(ch * L, L)] = jnp.minimum(row, 383)
    cp_t = pltpu.async_copy(tgt_hbm.at[tidx_v], t_v, sem_t)
    cp_r.wait()
    for ch in range(RPW // L):
        r = ridx_v[pl.ds(ch * L, L)]
        eidx_v[pl.ds(ch * L, L)] = lax.shift_right_logical(r, 1)
        midx_v[pl.ds(ch * L, L)] = lax.shift_right_logical(r, 7)
    cp_g = pltpu.async_copy(ev2_hbm.at[eidx_v], g_v, sem_g)
    cp_m = pltpu.async_copy(mean_hbm.at[midx_v], mean_v, sem_m)
    cp_x.wait()
    cp_g.wait()
    cp_m.wait()
    cp_t.wait()

    row3 = [iota * 3 + j for j in range(3)]
    rj = [plsc.load_gather(ridx_v, [row3[j]]) for j in range(3)]
    mt = [plsc.load_gather(mean_v, [row3[j], rj[j] & 127]) for j in range(3)]
    par = [(rj[j] & 1) * 64 for j in range(3)]
    toff = (base & 127) + iota * 3
    trow2 = lax.shift_right_logical(toff, 7)
    tcol = toff & 127

    for bc in range(B // 8):
        acc = [[mt[j] for _ in range(8)] for j in range(3)]
        for k in range(30):
            col = [plsc.load_gather(g_v, [row3[j], par[j] + k])
                   for j in range(3)]
            for b8 in range(8):
                b = bc * 8 + b8
                xv = plsc.load_gather(
                    x_v, [jnp.full((L,), 8 + b * KX + k, jnp.int32)])
                for j in range(3):
                    acc[j][b8] = acc[j][b8] + col[j] * xv
        for b8 in range(8):
            b = bc * 8 + b8
            tt = plsc.load_gather(t_v, [trow2 + 2 * b, tcol])
            d0 = acc[0][b8] - tt
            d1 = acc[1][b8] - tt
            d2 = acc[2][b8] - tt
            dist = _nsqrt(d0 * d0 + d1 * d1 + d2 * d2)
            part_v[b, :] = jnp.broadcast_to(jnp.sum(dist), (L,))

    pltpu.sync_copy(part_v, stage.at[c, s])
    plsc.subcore_barrier()

    @pl.when(s == 0)
    def _fold():
        cps = [pltpu.async_copy(stage.at[c, i], red_v.at[i], sem_g)
               for i in range(16)]
        for cp in cps:
            cp.wait()
        for b in range(B):
            acc_r = red_v[0, b, :]
            for i in range(1, 16):
                acc_r = acc_r + red_v[i, b, :]
            osum_v[b, :] = acc_r
        pltpu.sync_copy(osum_v, out_hbm.at[c])


def kernel(output, target, eigenVectors, mean, indices, outline, rightLineIdxs):
    ev2 = eigenVectors.reshape(-1, 128)
    tgt2 = target.reshape(-1, 128)
    mean2 = mean.reshape(-1, 128)
    part = _sc_loss(ev2, rightLineIdxs, output.reshape(-1), tgt2, mean2)
    return (part[0] + part[1])[:, 0]


# speedup vs baseline: 2.6823x; 1.0649x over previous
import functools

import jax
import jax.numpy as jnp
from jax import lax
from jax.experimental import pallas as pl
from jax.experimental.pallas import tpu as pltpu
from jax.experimental.pallas import tpu_sc as plsc

B = 32
KX = 30
NTRI = 512
NW = 32
TPW = NTRI // NW
RPW = 3 * TPW
L = 16

_mesh = plsc.VectorSubcoreMesh(core_axis_name="c", subcore_axis_name="s")


def _nsqrt(x):
    i = lax.bitcast_convert_type(x, jnp.int32)
    y = lax.bitcast_convert_type(
        lax.shift_right_arithmetic(i, 1) + jnp.int32(0x1FBD1DF5), jnp.float32)
    for _ in range(3):
        y = 0.5 * (y + x / y)
    return y


@functools.partial(
    pl.kernel,
    out_type=jax.ShapeDtypeStruct((2, B, L), jnp.float32),
    mesh=_mesh,
    compiler_params=pltpu.CompilerParams(needs_layout_passes=False,
                                         use_tc_tiling_on_sc=True),
    scratch_types=[
        pltpu.VMEM((RPW,), jnp.int32),
        pltpu.VMEM((RPW,), jnp.int32),
        pltpu.VMEM((RPW,), jnp.int32),
        pltpu.VMEM((RPW, 128), jnp.float32),
        pltpu.VMEM((RPW, 128), jnp.float32),
        pltpu.VMEM((8 + B * KX, ), jnp.float32),
        pltpu.VMEM((2 * B,), jnp.int32),
        pltpu.VMEM((2 * B, 128), jnp.float32),
        pltpu.VMEM((B, L), jnp.float32),
        pltpu.VMEM((16, B, L), jnp.float32),
        pltpu.VMEM((B, L), jnp.float32),
        pltpu.HBM((2, 16, B, L), jnp.float32),
        pltpu.SemaphoreType.DMA,
        pltpu.SemaphoreType.DMA,
        pltpu.SemaphoreType.DMA,
        pltpu.SemaphoreType.DMA,
    ],
)
def _sc_loss(ev2_hbm, rlt_hbm, x_hbm, tgt_hbm, mean_hbm, out_hbm,
             ridx_v, eidx_v, midx_v, g_v, mean_v, x_v, tidx_v, t_v, part_v,
             red_v, osum_v, stage, sem_g, sem_m, sem_t, sem_x):
    c = lax.axis_index("c")
    s = lax.axis_index("s")
    g = c * 16 + s
    base = g * RPW

    iota = lax.iota(jnp.int32, L)

    cp_r = pltpu.async_copy(rlt_hbm.at[pl.ds(base, RPW)], ridx_v, sem_m)
    cp_x = pltpu.async_copy(x_hbm, x_v.at[pl.ds(8, B * KX)], sem_x)
    trow = lax.shift_right_logical(base, 7)
    for ch in range(2 * B // L):
        n = iota + ch * L
        row = 12 * lax.shift_right_logical(n, 1) + trow + (n & 1)
        tidx_v[pl.ds(ch * L, L)] = jnp.minimum(row, 383)
    cp_t = pltpu.async_copy(tgt_hbm.at[tidx_v], t_v, sem_t)
    cp_r.wait()
    for ch in range(RPW // L):
        r = ridx_v[pl.ds(ch * L, L)]
        eidx_v[pl.ds(ch * L, L)] = lax.shift_right_logical(r, 2)
        midx_v[pl.ds(ch * L, L)] = lax.shift_right_logical(r, 7)
    cp_g = pltpu.async_copy(ev2_hbm.at[eidx_v], g_v, sem_g)
    cp_m = pltpu.async_copy(mean_hbm.at[midx_v], mean_v, sem_m)
    cp_x.wait()
    cp_g.wait()
    cp_m.wait()
    cp_t.wait()

    row3 = [iota * 3 + j for j in range(3)]
    rj = [plsc.load_gather(ridx_v, [row3[j]]) for j in range(3)]
    mt = [plsc.load_gather(mean_v, [row3[j], rj[j] & 127]) for j in range(3)]
    par = [(rj[j] & 3) * 32 for j in range(3)]
    toff = (base & 127) + iota * 3
    trow2 = lax.shift_right_logical(toff, 7)
    tcol = toff & 127

    for bc in range(B // 8):
        acc = [[mt[j] for _ in range(8)] for j in range(3)]
        for k in range(30):
            col = [plsc.load_gather(g_v, [row3[j], par[j] + k])
                   for j in range(3)]
            for b8 in range(8):
                b = bc * 8 + b8
                xv = plsc.load_gather(
                    x_v, [jnp.full((L,), 8 + b * KX + k, jnp.int32)])
                for j in range(3):
                    acc[j][b8] = acc[j][b8] + col[j] * xv
        for b8 in range(8):
            b = bc * 8 + b8
            tt = plsc.load_gather(t_v, [trow2 + 2 * b, tcol])
            d0 = acc[0][b8] - tt
            d1 = acc[1][b8] - tt
            d2 = acc[2][b8] - tt
            dist = _nsqrt(d0 * d0 + d1 * d1 + d2 * d2)
            part_v[b, :] = jnp.broadcast_to(jnp.sum(dist), (L,))

    pltpu.sync_copy(part_v, stage.at[c, s])
    plsc.subcore_barrier()

    @pl.when(s == 0)
    def _fold():
        cps = [pltpu.async_copy(stage.at[c, i], red_v.at[i], sem_g)
               for i in range(16)]
        for cp in cps:
            cp.wait()
        for b in range(B):
            acc_r = red_v[0, b, :]
            for i in range(1, 16):
                acc_r = acc_r + red_v[i, b, :]
            osum_v[b, :] = acc_r
        pltpu.sync_copy(osum_v, out_hbm.at[c])


def kernel(output, target, eigenVectors, mean, indices, outline, rightLineIdxs):
    ev2 = eigenVectors[:, :32].reshape(-1, 128)
    tgt2 = target.reshape(-1, 128)
    mean2 = mean.reshape(-1, 128)
    part = _sc_loss(ev2, rightLineIdxs, output.reshape(-1), tgt2, mean2)
    return (part[0] + part[1])[:, 0]
